# Initial kernel scaffold; baseline (speedup 1.0000x reference)
import functools

import jax
import jax.numpy as jnp
from jax import lax
from jax.experimental import pallas as pl
from jax.experimental.pallas import tpu as pltpu
from jax.experimental.pallas import tpu_sc as plsc

_NC = 2
_NS = 16
_NW = _NC * _NS


def _rsqrt_newton(s):
    i = lax.bitcast_convert_type(s, jnp.int32)
    i = jnp.int32(0x5F3759DF) - lax.shift_right_logical(i, 1)
    y = lax.bitcast_convert_type(i, jnp.float32)
    h = s * 0.5
    for _ in range(3):
        y = y * (1.5 - h * y * y)
    return y


@functools.lru_cache(maxsize=None)
def _make_pool_kernel(B, L, D, V):
    CB = 16
    RPC = CB * L
    PW = B // _NW
    NCH = PW // CB
    KD = D // 16
    mesh = plsc.VectorSubcoreMesh(core_axis_name="c", subcore_axis_name="s")

    @functools.partial(
        pl.kernel,
        mesh=mesh,
        out_type=jax.ShapeDtypeStruct((B, D), jnp.float32),
        scratch_types=[
            pltpu.VMEM((RPC,), jnp.int32),
            pltpu.VMEM((RPC, D), jnp.float32),
            pltpu.VMEM((CB, D), jnp.float32),
            pltpu.SemaphoreType.DMA,
        ],
    )
    def pool(x_hbm, table_hbm, out_hbm, idx_v, rows_v, pooled_v, sem):
        wid = lax.axis_index("s") * _NC + lax.axis_index("c")

        def batch_body(j, carry):
            accs = [jnp.zeros((16,), jnp.float32)] * KD
            for l in range(L):
                r = j * L + l
                vs = [rows_v[r, pl.ds(16 * k, 16)] for k in range(KD)]
                ss = vs[0] * vs[0]
                for k in range(1, KD):
                    ss = ss + vs[k] * vs[k]
                tot = jnp.sum(ss)
                sb = jnp.full((16,), tot, jnp.float32)
                scale = jnp.minimum(_rsqrt_newton(sb), 1.0)
                accs = [a + scale * v for a, v in zip(accs, vs)]
            inv = jnp.float32(1.0 / L)
            for k in range(KD):
                pooled_v[j, pl.ds(16 * k, 16)] = accs[k] * inv
            return carry

        def chunk_body(ci, carry):
            base_b = wid * PW + ci * CB
            base_r = base_b * L
            pltpu.sync_copy(x_hbm.at[pl.ds(base_r, RPC)], idx_v)
            pltpu.async_copy(table_hbm.at[idx_v], rows_v, sem).wait()
            lax.fori_loop(0, CB, batch_body, 0)
            pltpu.sync_copy(pooled_v, out_hbm.at[pl.ds(base_b, CB)])
            return carry

        lax.fori_loop(0, NCH, chunk_body, 0)

    return pool


def _mlp_body(xe_ref, w1_ref, b1_ref, w2_ref, b2_ref, fc1_ref, fc2_ref, pred_ref):
    x = xe_ref[...]
    h = lax.dot_general(x, w1_ref[...], (((1,), (1,)), ((), ())),
                        preferred_element_type=jnp.float32)
    h = jnp.maximum(h + b1_ref[...], 0.0)
    fc1_ref[...] = h
    z = jnp.sum(h * w2_ref[...], axis=1, keepdims=True) + b2_ref[...]
    fc2_ref[...] = z
    pred_ref[...] = 1.0 / (1.0 + jnp.exp(-z))


@functools.lru_cache(maxsize=None)
def _make_mlp(B, D, H, BT):
    grid = (B // BT,)
    return pl.pallas_call(
        _mlp_body,
        grid=grid,
        in_specs=[
            pl.BlockSpec((BT, D), lambda i: (i, 0)),
            pl.BlockSpec((H, D), lambda i: (0, 0)),
            pl.BlockSpec((1, H), lambda i: (0, 0)),
            pl.BlockSpec((1, H), lambda i: (0, 0)),
            pl.BlockSpec((1, 1), lambda i: (0, 0)),
        ],
        out_specs=[
            pl.BlockSpec((BT, H), lambda i: (i, 0)),
            pl.BlockSpec((BT, 1), lambda i: (i, 0)),
            pl.BlockSpec((BT, 1), lambda i: (i, 0)),
        ],
        out_shape=[
            jax.ShapeDtypeStruct((B, H), jnp.float32),
            jax.ShapeDtypeStruct((B, 1), jnp.float32),
            jax.ShapeDtypeStruct((B, 1), jnp.float32),
        ],
    )


def kernel(x, table, W1, b1, W2, b2):
    B, L = x.shape
    V, D = table.shape
    H = W1.shape[0]
    x_flat = x.reshape(B * L).astype(jnp.int32)
    x_embed = _make_pool_kernel(B, L, D, V)(x_flat, table)
    fc1, fc2, pred = _make_mlp(B, D, H, 1024)(
        x_embed, W1, b1.reshape(1, H), W2, b2.reshape(1, 1))
    return fc1, fc2, pred


# speedup vs baseline: 5.3334x; 5.3334x over previous
import functools

import jax
import jax.numpy as jnp
from jax import lax
from jax.experimental import pallas as pl
from jax.experimental.pallas import tpu as pltpu
from jax.experimental.pallas import tpu_sc as plsc

_NC = 2
_NS = 16
_NW = _NC * _NS


_GDN = lax.GatherDimensionNumbers(
    offset_dims=(), collapsed_slice_dims=(0,), start_index_map=(0,))


def _lane_shuffle(v, idx):
    return lax.gather(v, idx[:, None], dimension_numbers=_GDN,
                      slice_sizes=(1,),
                      mode=lax.GatherScatterMode.PROMISE_IN_BOUNDS)


def _lane_allsum(v):
    lanes = lax.iota(jnp.int32, 16)
    for sh in (1, 2, 4, 8):
        v = v + _lane_shuffle(v, lanes ^ sh)
    return v


def _rsqrt_newton(s):
    i = lax.bitcast_convert_type(s, jnp.int32)
    i = jnp.int32(0x5F3759DF) - lax.shift_right_logical(i, 1)
    y = lax.bitcast_convert_type(i, jnp.float32)
    h = s * 0.5
    for _ in range(3):
        y = y * (1.5 - h * y * y)
    return y


@functools.lru_cache(maxsize=None)
def _make_pool_kernel(B, L, D, V):
    CB = 16
    RPC = CB * L
    PW = B // _NW
    NCH = PW // CB
    KD = D // 16
    mesh = plsc.VectorSubcoreMesh(core_axis_name="c", subcore_axis_name="s")

    @functools.partial(
        pl.kernel,
        mesh=mesh,
        out_type=jax.ShapeDtypeStruct((B, D), jnp.float32),
        scratch_types=[
            pltpu.VMEM((RPC,), jnp.int32),
            pltpu.VMEM((RPC, D), jnp.float32),
            pltpu.VMEM((CB, D), jnp.float32),
            pltpu.SemaphoreType.DMA,
        ],
    )
    def pool(x_hbm, table_hbm, out_hbm, idx_v, rows_v, pooled_v, sem):
        wid = lax.axis_index("s") * _NC + lax.axis_index("c")

        def batch_body(j, carry):
            accs = [jnp.zeros((16,), jnp.float32)] * KD
            for l in range(L):
                r = j * L + l
                vs = [rows_v[r, pl.ds(16 * k, 16)] for k in range(KD)]
                ss = vs[0] * vs[0]
                for k in range(1, KD):
                    ss = ss + vs[k] * vs[k]
                sb = _lane_allsum(ss)
                scale = jnp.minimum(_rsqrt_newton(sb), 1.0)
                accs = [a + scale * v for a, v in zip(accs, vs)]
            inv = jnp.float32(1.0 / L)
            for k in range(KD):
                pooled_v[j, pl.ds(16 * k, 16)] = accs[k] * inv
            return carry

        def chunk_body(ci, carry):
            base_b = wid * PW + ci * CB
            base_r = base_b * L
            pltpu.sync_copy(x_hbm.at[pl.ds(base_r, RPC)], idx_v)
            pltpu.async_copy(table_hbm.at[idx_v], rows_v, sem).wait()
            lax.fori_loop(0, CB, batch_body, 0)
            pltpu.sync_copy(pooled_v, out_hbm.at[pl.ds(base_b, CB)])
            return carry

        lax.fori_loop(0, NCH, chunk_body, 0)

    return pool


def _mlp_body(xe_ref, w1_ref, b1_ref, w2_ref, b2_ref, fc1_ref, fc2_ref, pred_ref):
    x = xe_ref[...]
    h = lax.dot_general(x, w1_ref[...], (((1,), (1,)), ((), ())),
                        preferred_element_type=jnp.float32)
    h = jnp.maximum(h + b1_ref[...], 0.0)
    fc1_ref[...] = h
    z = jnp.sum(h * w2_ref[...], axis=1, keepdims=True) + b2_ref[...]
    fc2_ref[...] = z
    pred_ref[...] = 1.0 / (1.0 + jnp.exp(-z))


@functools.lru_cache(maxsize=None)
def _make_mlp(B, D, H, BT):
    grid = (B // BT,)
    return pl.pallas_call(
        _mlp_body,
        grid=grid,
        in_specs=[
            pl.BlockSpec((BT, D), lambda i: (i, 0)),
            pl.BlockSpec((H, D), lambda i: (0, 0)),
            pl.BlockSpec((1, H), lambda i: (0, 0)),
            pl.BlockSpec((1, H), lambda i: (0, 0)),
            pl.BlockSpec((1, 1), lambda i: (0, 0)),
        ],
        out_specs=[
            pl.BlockSpec((BT, H), lambda i: (i, 0)),
            pl.BlockSpec((BT, 1), lambda i: (i, 0)),
            pl.BlockSpec((BT, 1), lambda i: (i, 0)),
        ],
        out_shape=[
            jax.ShapeDtypeStruct((B, H), jnp.float32),
            jax.ShapeDtypeStruct((B, 1), jnp.float32),
            jax.ShapeDtypeStruct((B, 1), jnp.float32),
        ],
    )


def kernel(x, table, W1, b1, W2, b2):
    B, L = x.shape
    V, D = table.shape
    H = W1.shape[0]
    x_flat = x.reshape(B * L).astype(jnp.int32)
    x_embed = _make_pool_kernel(B, L, D, V)(x_flat, table)
    fc1, fc2, pred = _make_mlp(B, D, H, 1024)(
        x_embed, W1, b1.reshape(1, H), W2, b2.reshape(1, 1))
    return fc1, fc2, pred


# speedup vs baseline: 7.3598x; 1.3799x over previous
import functools

import jax
import jax.numpy as jnp
from jax import lax
from jax.experimental import pallas as pl
from jax.experimental.pallas import tpu as pltpu
from jax.experimental.pallas import tpu_sc as plsc

_NC = 2
_NS = 16
_NW = _NC * _NS


_GDN = lax.GatherDimensionNumbers(
    offset_dims=(), collapsed_slice_dims=(0,), start_index_map=(0,))


def _lane_shuffle(v, idx):
    return lax.gather(v, idx[:, None], dimension_numbers=_GDN,
                      slice_sizes=(1,),
                      mode=lax.GatherScatterMode.PROMISE_IN_BOUNDS)


def _lane_allsum(v):
    lanes = lax.iota(jnp.int32, 16)
    for sh in (1, 2, 4, 8):
        v = v + _lane_shuffle(v, lanes ^ sh)
    return v


def _rsqrt_newton(s):
    i = lax.bitcast_convert_type(s, jnp.int32)
    i = jnp.int32(0x5F3759DF) - lax.shift_right_logical(i, 1)
    y = lax.bitcast_convert_type(i, jnp.float32)
    h = s * 0.5
    for _ in range(2):
        y = y * (1.5 - h * y * y)
    return y


@functools.lru_cache(maxsize=None)
def _make_pool_kernel(B, L, D, V):
    CB = 16
    RPC = CB * L
    PW = B // _NW
    NCH = PW // CB
    KD = D // 16
    mesh = plsc.VectorSubcoreMesh(core_axis_name="c", subcore_axis_name="s")

    @functools.partial(
        pl.kernel,
        mesh=mesh,
        out_type=jax.ShapeDtypeStruct((B, D), jnp.float32),
        scratch_types=[
            pltpu.VMEM((RPC,), jnp.int32),
            pltpu.VMEM((RPC,), jnp.int32),
            pltpu.VMEM((RPC, D), jnp.float32),
            pltpu.VMEM((RPC, D), jnp.float32),
            pltpu.VMEM((CB, D), jnp.float32),
            pltpu.SemaphoreType.DMA,
            pltpu.SemaphoreType.DMA,
        ],
    )
    def pool(x_hbm, table_hbm, out_hbm, idx_v0, idx_v1, rows_v0, rows_v1,
             pooled_v, sem0, sem1):
        wid = lax.axis_index("s") * _NC + lax.axis_index("c")
        base_b0 = wid * PW

        def start_fetch(ci, idx_v, rows_v, sem):
            base_r = (base_b0 + ci * CB) * L
            pltpu.sync_copy(x_hbm.at[pl.ds(base_r, RPC)], idx_v)
            pltpu.async_copy(table_hbm.at[idx_v], rows_v, sem)

        def wait_fetch(idx_v, rows_v, sem):
            pltpu.make_async_copy(table_hbm.at[idx_v], rows_v, sem).wait()

        def compute_chunk(ci, rows_v):
            def batch_body(j, carry):
                accs = [jnp.zeros((16,), jnp.float32)] * KD
                for l in range(L):
                    r = j * L + l
                    vs = [rows_v[r, pl.ds(16 * k, 16)] for k in range(KD)]
                    ss = vs[0] * vs[0]
                    for k in range(1, KD):
                        ss = ss + vs[k] * vs[k]
                    sb = _lane_allsum(ss)
                    scale = jnp.minimum(_rsqrt_newton(sb), 1.0)
                    accs = [a + scale * v for a, v in zip(accs, vs)]
                inv = jnp.float32(1.0 / L)
                for k in range(KD):
                    pooled_v[j, pl.ds(16 * k, 16)] = accs[k] * inv
                return carry

            lax.fori_loop(0, CB, batch_body, 0)
            pltpu.sync_copy(pooled_v, out_hbm.at[pl.ds(base_b0 + ci * CB, CB)])

        start_fetch(0, idx_v0, rows_v0, sem0)

        def pair_body(p, carry):
            ci0 = 2 * p
            wait_fetch(idx_v0, rows_v0, sem0)
            start_fetch(ci0 + 1, idx_v1, rows_v1, sem1)
            compute_chunk(ci0, rows_v0)
            wait_fetch(idx_v1, rows_v1, sem1)

            @pl.when(p + 1 < NCH // 2)
            def _():
                start_fetch(ci0 + 2, idx_v0, rows_v0, sem0)

            compute_chunk(ci0 + 1, rows_v1)
            return carry

        lax.fori_loop(0, NCH // 2, pair_body, 0)

    return pool


def _mlp_body(xe_ref, w1_ref, b1_ref, w2_ref, b2_ref, fc1_ref, fc2_ref, pred_ref):
    x = xe_ref[...]
    h = lax.dot_general(x, w1_ref[...], (((1,), (1,)), ((), ())),
                        preferred_element_type=jnp.float32)
    h = jnp.maximum(h + b1_ref[...], 0.0)
    fc1_ref[...] = h
    z = jnp.sum(h * w2_ref[...], axis=1, keepdims=True) + b2_ref[...]
    fc2_ref[...] = z
    pred_ref[...] = 1.0 / (1.0 + jnp.exp(-z))


@functools.lru_cache(maxsize=None)
def _make_mlp(B, D, H, BT):
    grid = (B // BT,)
    return pl.pallas_call(
        _mlp_body,
        grid=grid,
        in_specs=[
            pl.BlockSpec((BT, D), lambda i: (i, 0)),
            pl.BlockSpec((H, D), lambda i: (0, 0)),
            pl.BlockSpec((1, H), lambda i: (0, 0)),
            pl.BlockSpec((1, H), lambda i: (0, 0)),
            pl.BlockSpec((1, 1), lambda i: (0, 0)),
        ],
        out_specs=[
            pl.BlockSpec((BT, H), lambda i: (i, 0)),
            pl.BlockSpec((BT, 1), lambda i: (i, 0)),
            pl.BlockSpec((BT, 1), lambda i: (i, 0)),
        ],
        out_shape=[
            jax.ShapeDtypeStruct((B, H), jnp.float32),
            jax.ShapeDtypeStruct((B, 1), jnp.float32),
            jax.ShapeDtypeStruct((B, 1), jnp.float32),
        ],
    )


def kernel(x, table, W1, b1, W2, b2):
    B, L = x.shape
    V, D = table.shape
    H = W1.shape[0]
    x_flat = x.reshape(B * L).astype(jnp.int32)
    x_embed = _make_pool_kernel(B, L, D, V)(x_flat, table)
    fc1, fc2, pred = _make_mlp(B, D, H, 1024)(
        x_embed, W1, b1.reshape(1, H), W2, b2.reshape(1, 1))
    return fc1, fc2, pred
